# baseline (speedup 1.0000x reference)
import functools

import jax
import jax.numpy as jnp
from jax import lax
from jax.experimental import pallas as pl
from jax.experimental.pallas import tpu as pltpu
from jax.experimental.pallas import tpu_sc as plsc

D = 32
GROUP = 128
BLK = 512
GPB = BLK // GROUP
L = 16
NC, NS = 2, 16
NW = NC * NS
TC = 256
TSTRIDE = 62496
TCHUNKS = 246


def kernel(input_tensor, table):
    B, H = input_tensor.shape
    V = table.shape[0]
    assert B == NW * BLK
    assert (NS - 1) * TSTRIDE + TCHUNKS * TC >= V and TCHUNKS * TC > TSTRIDE
    idx_t = input_tensor.T.astype(jnp.int32)
    table_t = table.T
    mesh = plsc.VectorSubcoreMesh(core_axis_name="c", subcore_axis_name="s")

    @functools.partial(
        pl.kernel,
        mesh=mesh,
        out_type=jax.ShapeDtypeStruct((H, D, B), jnp.float32),
        scratch_types=[
            pltpu.HBM((V, D), jnp.float32),
            pltpu.VMEM((H, BLK), jnp.int32),
            pltpu.VMEM((D, TC), jnp.float32),
            pltpu.VMEM((D, TC), jnp.float32),
            pltpu.VMEM((TC, D + 1), jnp.float32),
            pltpu.VMEM((TC, D + 1), jnp.float32),
            pltpu.VMEM((BLK, D), jnp.float32),
            pltpu.VMEM((BLK, D), jnp.float32),
            pltpu.VMEM((D, BLK + 1), jnp.float32),
            pltpu.VMEM((D, BLK + 1), jnp.float32),
            pltpu.SemaphoreType.DMA,
            pltpu.SemaphoreType.DMA,
            pltpu.SemaphoreType.DMA,
            pltpu.SemaphoreType.DMA,
        ],
        compiler_params=pltpu.CompilerParams(
            use_tc_tiling_on_sc=False, needs_layout_passes=False
        ),
    )
    def emb(idx_hbm, tabn_hbm, out_hbm, scr_hbm, idx_v,
            bn0, bn1, br0, br1, buf0, buf1, tb0, tb1, sg0, sg1, ss0, ss1):
        bns = (bn0, bn1)
        brs = (br0, br1)
        bufs = (buf0, buf1)
        tbs = (tb0, tb1)
        sem_g = (sg0, sg1)
        sem_s = (ss0, ss1)
        sid = lax.axis_index("s")
        wid = sid * NC + lax.axis_index("c")
        bbase = pl.multiple_of(wid * BLK, BLK)
        iota = lax.iota(jnp.int32, L)

        tbase = sid * TSTRIDE

        def p1_v0(i):
            return pl.multiple_of(
                tbase + jnp.minimum(i * TC, V - TC - tbase), 8
            )

        def p1_fire_read(i, p):
            pltpu.async_copy(
                tabn_hbm.at[:, pl.ds(p1_v0(i), TC)], bns[p], sem_g[p]
            )

        def p1_drain_read(p):
            pltpu.make_async_copy(
                tabn_hbm.at[:, pl.ds(0, TC)], bns[p], sem_g[p]
            ).wait()

        def p1_drain_write(p):
            pltpu.make_async_copy(
                brs[p].at[:, pl.ds(0, D)],
                scr_hbm.at[pl.ds(0, TC)],
                sem_s[p],
            ).wait()

        def p1_transpose(p):
            bn, br = bns[p], brs[p]

            def d_body(d, carry):
                d_vec = jnp.full((L,), d, jnp.int32)
                for v0 in range(0, TC, L):
                    plsc.store_scatter(
                        br, [iota + v0, d_vec], bn[d, pl.ds(v0, L)]
                    )
                return carry

            lax.fori_loop(0, D, d_body, 0)

        p1_fire_read(0, 0)

        def p1_step(m, carry):
            for p in range(2):
                i = 2 * m + p
                p1_drain_read(p)

                @pl.when(i + 1 < TCHUNKS)
                def _():
                    p1_fire_read(i + 1, 1 - p)

                @pl.when(i >= 2)
                def _():
                    p1_drain_write(p)

                p1_transpose(p)
                pltpu.async_copy(
                    brs[p].at[:, pl.ds(0, D)],
                    scr_hbm.at[pl.ds(p1_v0(i), TC)],
                    sem_s[p],
                )
            return carry

        lax.fori_loop(0, TCHUNKS // 2, p1_step, 0)
        p1_drain_write(0)
        p1_drain_write(1)

        plsc.subcore_barrier()

        pltpu.sync_copy(idx_hbm.at[:, pl.ds(bbase, BLK)], idx_v)

        def fire_gathers(h, p):
            for j in range(GPB):
                pltpu.async_copy(
                    scr_hbm.at[idx_v.at[h, pl.ds(j * GROUP, GROUP)]],
                    bufs[p].at[pl.ds(j * GROUP, GROUP)],
                    sem_g[p],
                )

        def drain_gathers(p):
            pltpu.make_async_copy(
                scr_hbm.at[pl.ds(0, BLK)], bufs[p], sem_g[p]
            ).wait()

        def drain_store(p):
            pltpu.make_async_copy(
                tbs[p].at[:, pl.ds(0, BLK)],
                out_hbm.at[0, :, pl.ds(0, BLK)],
                sem_s[p],
            ).wait()

        d_lo = iota
        d_hi = iota + L

        TUNROLL = 64

        def transpose(p):
            buf, tb = bufs[p], tbs[p]

            def e_body(eb, carry):
                for k in range(TUNROLL):
                    e = eb * TUNROLL + k
                    e_vec = jnp.full((L,), e, jnp.int32)
                    plsc.store_scatter(tb, [d_lo, e_vec], buf[e, pl.ds(0, L)])
                    plsc.store_scatter(tb, [d_hi, e_vec], buf[e, pl.ds(L, L)])
                return carry

            lax.fori_loop(0, BLK // TUNROLL, e_body, 0)

        fire_gathers(0, 0)

        def step(m, carry):
            for p in range(2):
                h = 2 * m + p
                drain_gathers(p)

                @pl.when(h + 1 < H)
                def _():
                    fire_gathers(h + 1, 1 - p)

                @pl.when(h >= 2)
                def _():
                    drain_store(p)

                transpose(p)
                pltpu.async_copy(
                    tbs[p].at[:, pl.ds(0, BLK)],
                    out_hbm.at[h, :, pl.ds(bbase, BLK)],
                    sem_s[p],
                )
            return carry

        lax.fori_loop(0, H // 2, step, 0)
        drain_store((H - 1) % 2)
        drain_store((H - 2) % 2)

    out = emb(idx_t, table_t)
    return out.transpose(2, 0, 1)


# speedup vs baseline: 4.1705x; 4.1705x over previous
import functools

import jax
import jax.numpy as jnp
from jax import lax
from jax.experimental import pallas as pl
from jax.experimental.pallas import tpu as pltpu
from jax.experimental.pallas import tpu_sc as plsc

D = 32
GROUP = 128
BLK = 512
GPB = BLK // GROUP
L = 16
NC, NS = 2, 16
NW = NC * NS


def kernel(input_tensor, table):
    B, H = input_tensor.shape
    V = table.shape[0]
    assert B == NW * BLK
    idx_t = input_tensor.T.astype(jnp.int32)
    mesh = plsc.VectorSubcoreMesh(core_axis_name="c", subcore_axis_name="s")

    @functools.partial(
        pl.kernel,
        mesh=mesh,
        out_type=jax.ShapeDtypeStruct((H, D, B), jnp.float32),
        scratch_types=[
            pltpu.VMEM((H, BLK), jnp.int32),
            pltpu.VMEM((BLK, D), jnp.float32),
            pltpu.VMEM((BLK, D), jnp.float32),
            pltpu.VMEM((D, BLK + 1), jnp.float32),
            pltpu.VMEM((D, BLK + 1), jnp.float32),
            pltpu.SemaphoreType.DMA,
            pltpu.SemaphoreType.DMA,
            pltpu.SemaphoreType.DMA,
            pltpu.SemaphoreType.DMA,
        ],
        compiler_params=pltpu.CompilerParams(
            use_tc_tiling_on_sc=False, needs_layout_passes=False
        ),
    )
    def emb(idx_hbm, table_hbm, out_hbm, idx_v, buf0, buf1, tb0, tb1,
            sg0, sg1, ss0, ss1):
        bufs = (buf0, buf1)
        tbs = (tb0, tb1)
        sem_g = (sg0, sg1)
        sem_s = (ss0, ss1)
        wid = lax.axis_index("s") * NC + lax.axis_index("c")
        bbase = pl.multiple_of(wid * BLK, BLK)

        pltpu.sync_copy(idx_hbm.at[:, pl.ds(bbase, BLK)], idx_v)

        iota = lax.iota(jnp.int32, L)

        def fire_gathers(h, p):
            for j in range(GPB):
                pltpu.async_copy(
                    table_hbm.at[idx_v.at[h, pl.ds(j * GROUP, GROUP)]],
                    bufs[p].at[pl.ds(j * GROUP, GROUP)],
                    sem_g[p],
                )

        def drain_gathers(p):
            pltpu.make_async_copy(
                table_hbm.at[pl.ds(0, BLK)], bufs[p], sem_g[p]
            ).wait()

        def drain_store(p):
            pltpu.make_async_copy(
                tbs[p].at[:, pl.ds(0, BLK)],
                out_hbm.at[0, :, pl.ds(0, BLK)],
                sem_s[p],
            ).wait()

        d_lo = iota
        d_hi = iota + L

        def transpose(p):
            buf, tb = bufs[p], tbs[p]
            for e in range(BLK):
                e_vec = jnp.full((L,), e, jnp.int32)
                plsc.store_scatter(tb, [d_lo, e_vec], buf[e, pl.ds(0, L)])
                plsc.store_scatter(tb, [d_hi, e_vec], buf[e, pl.ds(L, L)])

        fire_gathers(0, 0)

        def step(m, carry):
            for p in range(2):
                h = 2 * m + p
                drain_gathers(p)

                @pl.when(h + 1 < H)
                def _():
                    fire_gathers(h + 1, 1 - p)

                @pl.when(h >= 2)
                def _():
                    drain_store(p)

                transpose(p)
                pltpu.async_copy(
                    tbs[p].at[:, pl.ds(0, BLK)],
                    out_hbm.at[h, :, pl.ds(bbase, BLK)],
                    sem_s[p],
                )
            return carry

        lax.fori_loop(0, H // 2, step, 0)
        drain_store((H - 1) % 2)
        drain_store((H - 2) % 2)

    out = emb(idx_t, table)
    return out.transpose(2, 0, 1)
